# baseline (speedup 1.0000x reference)
import functools

import jax
import jax.numpy as jnp
from jax import lax
from jax.experimental import pallas as pl
from jax.experimental.pallas import tpu as pltpu
from jax.experimental.pallas import tpu_sc as plsc

B, S, C = 128, 32, 8192
R = B * S
NEG_INF = float("-inf")
_LN2 = 0.6931471805599453
_SQRT2 = 1.4142135623730951
NW = 32
BPW = B // NW
UN = 8


def _vlog16(x):
    bits = plsc.bitcast(x, jnp.int32)
    e = jax.lax.shift_right_logical(bits, 23) - 127
    mant = jax.lax.bitwise_or(
        jax.lax.bitwise_and(bits, 0x007FFFFF), 0x3F800000
    )
    m = plsc.bitcast(mant, jnp.float32)
    big = m > _SQRT2
    m = jnp.where(big, m * 0.5, m)
    ef = e.astype(jnp.float32) + jnp.where(big, 1.0, 0.0)
    t = (m - 1.0) / (m + 1.0)
    t2 = t * t
    p = 2.0 * t * (1.0 + t2 * (1.0 / 3.0 + t2 * (1.0 / 5.0 + t2 * (1.0 / 7.0))))
    return ef * _LN2 + p


def _worker_id():
    return lax.axis_index("s") * 2 + lax.axis_index("c")


def _sc_body(logits_hbm, idx_hbm, out_hbm, idxrow, rowbuf):
    wid = _worker_id()
    lane = lax.iota(jnp.int32, 16)
    ninf16 = jnp.full((16,), NEG_INF, jnp.float32)

    for b_loc in range(BPW):
        b = wid * BPW + b_loc
        pltpu.sync_copy(idx_hbm.at[b], idxrow)
        idx_lo = idxrow[pl.ds(0, 16)]
        idx_hi = idxrow[pl.ds(16, 16)]

        def row_body(i, _, b=b, idx_lo=idx_lo, idx_hi=idx_hi):
            row = b * S + i
            pltpu.sync_copy(logits_hbm.at[row], rowbuf)
            v0 = rowbuf[pl.ds(0, 16)]
            rowbuf[pl.ds(0, 16)] = jnp.where(lane < 2, NEG_INF, v0)
            plsc.store_scatter(rowbuf, [idx_lo], ninf16, mask=lane <= i)
            plsc.store_scatter(rowbuf, [idx_hi], ninf16, mask=(lane + 16) <= i)

            @plsc.parallel_loop(0, C, 16 * UN, carry=(ninf16,) * UN)
            def p1(k, accs):
                return tuple(
                    jnp.maximum(accs[t], rowbuf[pl.ds(k + 16 * t, 16)])
                    for t in range(UN)
                )

            acc = p1
            red = list(acc)
            while len(red) > 1:
                red = [jnp.maximum(red[2 * j], red[2 * j + 1])
                       for j in range(len(red) // 2)]
            mv = jnp.full((16,), lax.reduce_max(red[0], axes=(0,)))

            z16 = jnp.zeros((16,), jnp.float32)

            @plsc.parallel_loop(0, C, 16 * UN, carry=(z16,) * UN)
            def p2(k, accs):
                return tuple(
                    accs[t] + jnp.exp(rowbuf[pl.ds(k + 16 * t, 16)] - mv)
                    for t in range(UN)
                )

            red = list(p2)
            while len(red) > 1:
                red = [red[2 * j] + red[2 * j + 1]
                       for j in range(len(red) // 2)]
            sv = jnp.full((16,), lax.reduce_sum(red[0], axes=(0,)))
            cv = mv + _vlog16(sv)

            @plsc.parallel_loop(0, C, 16 * UN)
            def p3(k):
                for t in range(UN):
                    sl = pl.ds(k + 16 * t, 16)
                    rowbuf[sl] = rowbuf[sl] - cv

            pltpu.sync_copy(rowbuf, out_hbm.at[row])
            return 0

        lax.fori_loop(0, S, row_body, 0)


def kernel(logits, tgt_in_idx):
    B, S, C = logits.shape
    R = B * S
    logits2 = logits.reshape(R, C)
    run = pl.kernel(
        _sc_body,
        out_type=jax.ShapeDtypeStruct((R, C), jnp.float32),
        mesh=plsc.VectorSubcoreMesh(core_axis_name="c", subcore_axis_name="s",
                                    num_cores=2, num_subcores=16),
        scratch_types=[
            pltpu.VMEM((S,), jnp.int32),
            pltpu.VMEM((C,), jnp.float32),
        ],
        compiler_params=pltpu.CompilerParams(needs_layout_passes=False),
    )
    return run(logits2, tgt_in_idx).reshape(B, S, C)


# speedup vs baseline: 1.0911x; 1.0911x over previous
import jax
import jax.numpy as jnp
from jax import lax
from jax.experimental import pallas as pl
from jax.experimental.pallas import tpu as pltpu
from jax.experimental.pallas import tpu_sc as plsc

B, S, C = 128, 32, 8192
B_SC = 32
B_TC = B - B_SC
NEG_INF = float("-inf")
_LN2 = 0.6931471805599453
_SQRT2 = 1.4142135623730951
NW = 32
UN = 8


def _vlog16(x):
    bits = plsc.bitcast(x, jnp.int32)
    e = lax.shift_right_logical(bits, 23) - 127
    mant = lax.bitwise_or(lax.bitwise_and(bits, 0x007FFFFF), 0x3F800000)
    m = plsc.bitcast(mant, jnp.float32)
    big = m > _SQRT2
    m = jnp.where(big, m * 0.5, m)
    ef = e.astype(jnp.float32) + jnp.where(big, 1.0, 0.0)
    t = (m - 1.0) / (m + 1.0)
    t2 = t * t
    p = 2.0 * t * (1.0 + t2 * (1.0 / 3.0 + t2 * (1.0 / 5.0 + t2 * (1.0 / 7.0))))
    return ef * _LN2 + p


def _worker_id():
    return lax.axis_index("s") * 2 + lax.axis_index("c")


def _sc_body(logits_hbm, idx_hbm, out_hbm, idxrow, rowbuf):
    bpw = B_SC // NW
    wid = _worker_id()
    lane = lax.iota(jnp.int32, 16)
    ninf16 = jnp.full((16,), NEG_INF, jnp.float32)

    for b_loc in range(bpw):
        b = wid * bpw + b_loc
        pltpu.sync_copy(idx_hbm.at[b], idxrow)
        idx_lo = idxrow[pl.ds(0, 16)]
        idx_hi = idxrow[pl.ds(16, 16)]

        def row_body(i, _, b=b, idx_lo=idx_lo, idx_hi=idx_hi):
            row = b * S + i
            pltpu.sync_copy(logits_hbm.at[row], rowbuf)
            v0 = rowbuf[pl.ds(0, 16)]
            rowbuf[pl.ds(0, 16)] = jnp.where(lane < 2, NEG_INF, v0)
            plsc.store_scatter(rowbuf, [idx_lo], ninf16, mask=lane <= i)
            plsc.store_scatter(rowbuf, [idx_hi], ninf16, mask=(lane + 16) <= i)

            @plsc.parallel_loop(0, C, 16 * UN, carry=(ninf16,) * UN)
            def p1(k, accs):
                return tuple(
                    jnp.maximum(accs[t], rowbuf[pl.ds(k + 16 * t, 16)])
                    for t in range(UN)
                )

            red = list(p1)
            while len(red) > 1:
                red = [jnp.maximum(red[2 * j], red[2 * j + 1])
                       for j in range(len(red) // 2)]
            mv = jnp.full((16,), lax.reduce_max(red[0], axes=(0,)))

            z16 = jnp.zeros((16,), jnp.float32)

            @plsc.parallel_loop(0, C, 16 * UN, carry=(z16,) * UN)
            def p2(k, accs):
                return tuple(
                    accs[t] + jnp.exp(rowbuf[pl.ds(k + 16 * t, 16)] - mv)
                    for t in range(UN)
                )

            red = list(p2)
            while len(red) > 1:
                red = [red[2 * j] + red[2 * j + 1]
                       for j in range(len(red) // 2)]
            sv = jnp.full((16,), lax.reduce_sum(red[0], axes=(0,)))
            cv = mv + _vlog16(sv)

            @plsc.parallel_loop(0, C, 16 * UN)
            def p3(k):
                for t in range(UN):
                    sl = pl.ds(k + 16 * t, 16)
                    rowbuf[sl] = rowbuf[sl] - cv

            pltpu.sync_copy(rowbuf, out_hbm.at[row])
            return 0

        lax.fori_loop(0, S, row_body, 0)


def _sc_call(logits_sc, idx_sc):
    run = pl.kernel(
        _sc_body,
        out_type=jax.ShapeDtypeStruct((B_SC * S, C), jnp.float32),
        mesh=plsc.VectorSubcoreMesh(core_axis_name="c", subcore_axis_name="s",
                                    num_cores=2, num_subcores=16),
        scratch_types=[
            pltpu.VMEM((S,), jnp.int32),
            pltpu.VMEM((C,), jnp.float32),
        ],
        compiler_params=pltpu.CompilerParams(needs_layout_passes=False),
    )
    return run(logits_sc.reshape(B_SC * S, C), idx_sc).reshape(B_SC, S, C)


def _tc_body(idx_ref, x_ref, o_ref):
    x = x_ref[0]
    idx = idx_ref[0]
    cand = lax.broadcasted_iota(jnp.int32, (S, C), 1)
    eq = (cand == idx).astype(jnp.float32)
    row = lax.broadcasted_iota(jnp.int32, (S, S), 0)
    col = lax.broadcasted_iota(jnp.int32, (S, S), 1)
    tril = (row >= col).astype(jnp.float32)
    counts = jnp.dot(tril, eq, preferred_element_type=jnp.float32)
    mask = (counts > 0.0) | (cand < 2)
    masked = jnp.where(mask, NEG_INF, x)
    m = jnp.max(masked, axis=1, keepdims=True)
    s = jnp.sum(jnp.exp(masked - m), axis=1, keepdims=True)
    o_ref[0] = masked - (m + jnp.log(s))


def _tc_call(logits_tc, idx_tc):
    nb = logits_tc.shape[0]
    return pl.pallas_call(
        _tc_body,
        grid=(nb,),
        in_specs=[
            pl.BlockSpec((1, S, 1), lambda b: (b, 0, 0)),
            pl.BlockSpec((1, S, C), lambda b: (b, 0, 0)),
        ],
        out_specs=pl.BlockSpec((1, S, C), lambda b: (b, 0, 0)),
        out_shape=jax.ShapeDtypeStruct((nb, S, C), jnp.float32),
        compiler_params=pltpu.CompilerParams(
            dimension_semantics=("arbitrary",),
        ),
    )(idx_tc[:, :, None], logits_tc)


def kernel(logits, tgt_in_idx):
    out_tc = _tc_call(logits[:B_TC], tgt_in_idx[:B_TC])
    out_sc = _sc_call(logits[B_TC:], tgt_in_idx[B_TC:])
    return jnp.concatenate([out_tc, out_sc], axis=0)


# speedup vs baseline: 2.9443x; 2.6985x over previous
import jax
import jax.numpy as jnp
from jax import lax
from jax.experimental import pallas as pl
from jax.experimental.pallas import tpu as pltpu

B, S, C = 128, 32, 8192
R = B * S
RB = 64
NEG_INF = float("-inf")


def _tc_body(idx_ref, x_ref, o_ref):
    x = x_ref[...]
    idxcol = idx_ref[...]
    cand = lax.broadcasted_iota(jnp.int32, (RB, C), 1)
    eq = (cand == idxcol).astype(jnp.float32)
    row = lax.broadcasted_iota(jnp.int32, (RB, RB), 0)
    col = lax.broadcasted_iota(jnp.int32, (RB, RB), 1)
    same_b = (row // S) == (col // S)
    tril = (same_b & (row >= col)).astype(jnp.float32)
    counts = jnp.dot(tril, eq, preferred_element_type=jnp.float32)
    mask = (counts > 0.0) | (cand < 2)
    masked = jnp.where(mask, NEG_INF, x)
    m = jnp.max(masked, axis=1, keepdims=True)
    s = jnp.sum(jnp.exp(masked - m), axis=1, keepdims=True)
    o_ref[...] = masked - (m + jnp.log(s))


def kernel(logits, tgt_in_idx):
    x2 = logits.reshape(R, C)
    idx2 = tgt_in_idx.reshape(R, 1)
    out = pl.pallas_call(
        _tc_body,
        grid=(R // RB,),
        in_specs=[
            pl.BlockSpec((RB, 1), lambda r: (r, 0)),
            pl.BlockSpec((RB, C), lambda r: (r, 0)),
        ],
        out_specs=pl.BlockSpec((RB, C), lambda r: (r, 0)),
        out_shape=jax.ShapeDtypeStruct((R, C), jnp.float32),
        compiler_params=pltpu.CompilerParams(
            dimension_semantics=("arbitrary",),
        ),
    )(idx2, x2)
    return out.reshape(B, S, C)


# speedup vs baseline: 3.5545x; 1.2073x over previous
import jax
import jax.numpy as jnp
from jax import lax
from jax.experimental import pallas as pl
from jax.experimental.pallas import tpu as pltpu

B, S, C = 128, 32, 8192
R = B * S
RB = 128
NEG_INF = float("-inf")


def _tc_body(idx_ref, x_ref, o_ref):
    x = x_ref[...]
    idxcol = idx_ref[...]
    cand = lax.broadcasted_iota(jnp.int32, (RB, C), 1)
    eq = (cand == idxcol).astype(jnp.float32)
    row = lax.broadcasted_iota(jnp.int32, (RB, RB), 0)
    col = lax.broadcasted_iota(jnp.int32, (RB, RB), 1)
    same_b = (row // S) == (col // S)
    tril = (same_b & (row >= col)).astype(jnp.float32)
    counts = jnp.dot(tril, eq, preferred_element_type=jnp.float32)
    mask = (counts > 0.0) | (cand < 2)
    masked = jnp.where(mask, NEG_INF, x)
    m = jnp.max(masked, axis=1, keepdims=True)
    s = jnp.sum(jnp.exp(masked - m), axis=1, keepdims=True)
    o_ref[...] = masked - (m + jnp.log(s))


def kernel(logits, tgt_in_idx):
    x2 = logits.reshape(R, C)
    idx2 = tgt_in_idx.reshape(R, 1)
    out = pl.pallas_call(
        _tc_body,
        grid=(R // RB,),
        in_specs=[
            pl.BlockSpec((RB, 1), lambda r: (r, 0)),
            pl.BlockSpec((RB, C), lambda r: (r, 0)),
        ],
        out_specs=pl.BlockSpec((RB, C), lambda r: (r, 0)),
        out_shape=jax.ShapeDtypeStruct((R, C), jnp.float32),
        compiler_params=pltpu.CompilerParams(
            dimension_semantics=("arbitrary",),
        ),
    )(idx2, x2)
    return out.reshape(B, S, C)


# speedup vs baseline: 3.7839x; 1.0645x over previous
import jax
import jax.numpy as jnp
from jax import lax
from jax.experimental import pallas as pl
from jax.experimental.pallas import tpu as pltpu

B, S, C = 128, 32, 8192
R = B * S
RB = 256
NEG_INF = float("-inf")


def _tc_body(idx_ref, x_ref, o_ref):
    x = x_ref[...]
    idxcol = idx_ref[...]
    cand = lax.broadcasted_iota(jnp.int32, (RB, C), 1)
    eq = (cand == idxcol).astype(jnp.float32)
    row = lax.broadcasted_iota(jnp.int32, (RB, RB), 0)
    col = lax.broadcasted_iota(jnp.int32, (RB, RB), 1)
    same_b = (row // S) == (col // S)
    tril = (same_b & (row >= col)).astype(jnp.float32)
    counts = jnp.dot(tril, eq, preferred_element_type=jnp.float32)
    mask = (counts > 0.0) | (cand < 2)
    masked = jnp.where(mask, NEG_INF, x)
    m = jnp.max(masked, axis=1, keepdims=True)
    s = jnp.sum(jnp.exp(masked - m), axis=1, keepdims=True)
    o_ref[...] = masked - (m + jnp.log(s))


def kernel(logits, tgt_in_idx):
    x2 = logits.reshape(R, C)
    idx2 = tgt_in_idx.reshape(R, 1)
    out = pl.pallas_call(
        _tc_body,
        grid=(R // RB,),
        in_specs=[
            pl.BlockSpec((RB, 1), lambda r: (r, 0)),
            pl.BlockSpec((RB, C), lambda r: (r, 0)),
        ],
        out_specs=pl.BlockSpec((RB, C), lambda r: (r, 0)),
        out_shape=jax.ShapeDtypeStruct((R, C), jnp.float32),
        compiler_params=pltpu.CompilerParams(
            dimension_semantics=("arbitrary",),
        ),
    )(idx2, x2)
    return out.reshape(B, S, C)
